# Initial kernel scaffold; baseline (speedup 1.0000x reference)
import jax
import jax.numpy as jnp
from jax.experimental import pallas as pl


def kernel(features, labels, centers):
    raise NotImplementedError("write your pallas kernel here")


# speedup vs baseline: 1.8458x; 1.8458x over previous
import functools

import jax
import jax.numpy as jnp
from jax import lax
from jax.experimental import pallas as pl
from jax.experimental.pallas import tpu as pltpu
from jax.experimental.pallas import tpu_sc as plsc

_B = 16384
_D = 64
_C = 1000
_CP = 1008
_L = 16
_NC = 2
_NS = 16
_NW = _NC * _NS
_BW = _B // _NW
_BH = _B // _NS


def _body(features_hbm, labels_hbm, centers_hbm, out_hbm,
          cent_v, feat_v, lab_hist_v, lab_my_v, hist_v, hist_all_v,
          tot_v, inv_v, out_v, part_v,
          hist_stage_s, part_stage_s, sem_c, sem_f):
    cid = lax.axis_index("c")
    sid = lax.axis_index("s")
    wid = cid * _NS + sid

    cp_c = pltpu.async_copy(centers_hbm, cent_v, sem_c)
    cp_f = pltpu.async_copy(
        features_hbm.at[pl.ds(wid * _BW, _BW)], feat_v, sem_f)

    pltpu.sync_copy(labels_hbm.at[pl.ds(sid * _BH, _BH)], lab_hist_v)

    def zero_hist(k, _):
        hist_v[pl.ds(k * _L, _L)] = jnp.zeros((_L,), jnp.float32)
        return 0
    lax.fori_loop(0, _CP // _L, zero_hist, 0)

    ones = jnp.ones((_L,), jnp.float32)

    def hist_step(i, _):
        idx = lab_hist_v[pl.ds(i * _L, _L)]
        plsc.addupdate_scatter(hist_v, [idx], ones)
        return 0
    lax.fori_loop(0, _BH // _L, hist_step, 0)

    pltpu.sync_copy(hist_v, hist_stage_s.at[sid])
    plsc.subcore_barrier()
    pltpu.sync_copy(hist_stage_s, hist_all_v)

    def sum_hist(k, _):
        sl = pl.ds(k * _L, _L)
        acc = hist_all_v[0, sl]
        def add_row(r, a):
            return a + hist_all_v[r, sl]
        tot_v[sl] = lax.fori_loop(1, _NS, add_row, acc)
        return 0
    lax.fori_loop(0, _CP // _L, sum_hist, 0)

    def inv_step(k, _):
        sl = pl.ds(k * _L, _L)
        n = tot_v[sl]
        inv_v[sl] = jnp.where(n > 0.0, 1.0 / n, 0.0)
        return 0
    lax.fori_loop(0, _CP // _L, inv_step, 0)

    cp_c.wait()
    cp_f.wait()
    pltpu.sync_copy(labels_hbm.at[pl.ds(wid * _BW, _BW)], lab_my_v)

    def sample_step(i, acc):
        rows = i * _L + lax.iota(jnp.int32, _L)
        idx = lab_my_v[pl.ds(i * _L, _L)]
        inv16 = plsc.load_gather(inv_v, [idx])
        ps = jnp.zeros((_L,), jnp.float32)
        for k in range(_D):
            kcol = jnp.full((_L,), k, jnp.int32)
            f = plsc.load_gather(feat_v, [rows, kcol])
            c = plsc.load_gather(cent_v, [idx, kcol])
            dlt = f - c
            ps = ps + dlt * dlt
        return acc + ps * inv16
    acc = lax.fori_loop(0, _BW // _L, sample_step,
                        jnp.zeros((_L,), jnp.float32))

    out_v[...] = acc
    pltpu.sync_copy(out_v, part_stage_s.at[sid])
    plsc.subcore_barrier()

    @pl.when(sid == 0)
    def _():
        pltpu.sync_copy(part_stage_s, part_v)
        def add_part(r, a):
            return a + part_v[r, :]
        tot = lax.fori_loop(1, _NS, add_part, part_v[0, :])
        out_v[...] = tot * (1.0 / (_D * _B))
        pltpu.sync_copy(out_v, out_hbm.at[cid])


@jax.jit
def _center_loss_sc(features, labels, centers):
    mesh = plsc.VectorSubcoreMesh(core_axis_name="c", subcore_axis_name="s")
    out = pl.kernel(
        _body,
        out_type=jax.ShapeDtypeStruct((_NC, _L), jnp.float32),
        mesh=mesh,
        compiler_params=pltpu.CompilerParams(
            needs_layout_passes=False, use_tc_tiling_on_sc=False),
        scratch_types=[
            pltpu.VMEM((_C, _D), jnp.float32),
            pltpu.VMEM((_BW, _D), jnp.float32),
            pltpu.VMEM((_BH,), jnp.int32),
            pltpu.VMEM((_BW,), jnp.int32),
            pltpu.VMEM((_CP,), jnp.float32),
            pltpu.VMEM((_NS, _CP), jnp.float32),
            pltpu.VMEM((_CP,), jnp.float32),
            pltpu.VMEM((_CP,), jnp.float32),
            pltpu.VMEM((_L,), jnp.float32),
            pltpu.VMEM((_NS, _L), jnp.float32),
            pltpu.VMEM_SHARED((_NS, _CP), jnp.float32),
            pltpu.VMEM_SHARED((_NS, _L), jnp.float32),
            pltpu.SemaphoreType.DMA,
            pltpu.SemaphoreType.DMA,
        ],
    )(features, labels, centers)
    return jnp.sum(out)


def kernel(features, labels, centers):
    labels = labels.reshape(-1).astype(jnp.int32)
    return _center_loss_sc(features, labels, centers)


# speedup vs baseline: 1.8493x; 1.0019x over previous
import functools

import jax
import jax.numpy as jnp
from jax import lax
from jax.experimental import pallas as pl
from jax.experimental.pallas import tpu as pltpu
from jax.experimental.pallas import tpu_sc as plsc

_B = 16384
_D = 64
_C = 1000
_CP = 1008
_L = 16
_NC = 2
_NS = 16
_NW = _NC * _NS
_BW = _B // _NW
_BH = _B // _NS


def _body(features_hbm, labels_hbm, centers_hbm, out_hbm,
          cent_v, feat_v, lab_hist_v, lab_my_v, hist_v, hist_all_v,
          tot_v, inv_v, out_v, part_v,
          hist_stage_s, part_stage_s, sem_c, sem_f):
    cid = lax.axis_index("c")
    sid = lax.axis_index("s")
    wid = cid * _NS + sid

    cp_c = pltpu.async_copy(centers_hbm, cent_v, sem_c)
    cp_f = pltpu.async_copy(
        features_hbm.at[pl.ds(wid * _BW, _BW)], feat_v, sem_f)

    with jax.named_scope("ph1_labels_dma"):
        pltpu.sync_copy(labels_hbm.at[pl.ds(sid * _BH, _BH)], lab_hist_v)

    with jax.named_scope("ph1_hist"):
        def zero_hist(k, _):
            hist_v[pl.ds(k * _L, _L)] = jnp.zeros((_L,), jnp.float32)
            return 0
        lax.fori_loop(0, _CP // _L, zero_hist, 0)

        ones = jnp.ones((_L,), jnp.float32)

        def hist_step(i, _):
            idx = lab_hist_v[pl.ds(i * _L, _L)]
            plsc.addupdate_scatter(hist_v, [idx], ones)
            return 0
        lax.fori_loop(0, _BH // _L, hist_step, 0)

    with jax.named_scope("ph1_allreduce"):
        pltpu.sync_copy(hist_v, hist_stage_s.at[sid])
        plsc.subcore_barrier()
        pltpu.sync_copy(hist_stage_s, hist_all_v)

        def sum_hist(k, _):
            sl = pl.ds(k * _L, _L)
            acc = hist_all_v[0, sl]
            def add_row(r, a):
                return a + hist_all_v[r, sl]
            tot_v[sl] = lax.fori_loop(1, _NS, add_row, acc)
            return 0
        lax.fori_loop(0, _CP // _L, sum_hist, 0)

        def inv_step(k, _):
            sl = pl.ds(k * _L, _L)
            n = tot_v[sl]
            inv_v[sl] = jnp.where(n > 0.0, 1.0 / n, 0.0)
            return 0
        lax.fori_loop(0, _CP // _L, inv_step, 0)

    with jax.named_scope("ph2_dma_wait"):
        cp_c.wait()
        cp_f.wait()
        pltpu.sync_copy(labels_hbm.at[pl.ds(wid * _BW, _BW)], lab_my_v)

    with jax.named_scope("ph2_main"):
        def sample_step(i, acc):
            rows = i * _L + lax.iota(jnp.int32, _L)
            idx = lab_my_v[pl.ds(i * _L, _L)]
            inv16 = plsc.load_gather(inv_v, [idx])
            ps = jnp.zeros((_L,), jnp.float32)
            for k in range(_D):
                kcol = jnp.full((_L,), k, jnp.int32)
                f = plsc.load_gather(feat_v, [rows, kcol])
                c = plsc.load_gather(cent_v, [idx, kcol])
                dlt = f - c
                ps = ps + dlt * dlt
            return acc + ps * inv16
        acc = lax.fori_loop(0, _BW // _L, sample_step,
                            jnp.zeros((_L,), jnp.float32))

    out_v[...] = acc
    pltpu.sync_copy(out_v, part_stage_s.at[sid])
    plsc.subcore_barrier()

    @pl.when(sid == 0)
    def _():
        pltpu.sync_copy(part_stage_s, part_v)
        def add_part(r, a):
            return a + part_v[r, :]
        tot = lax.fori_loop(1, _NS, add_part, part_v[0, :])
        out_v[...] = tot * (1.0 / (_D * _B))
        pltpu.sync_copy(out_v, out_hbm.at[cid])


@jax.jit
def _center_loss_sc(features, labels, centers):
    mesh = plsc.VectorSubcoreMesh(core_axis_name="c", subcore_axis_name="s")
    out = pl.kernel(
        _body,
        out_type=jax.ShapeDtypeStruct((_NC, _L), jnp.float32),
        mesh=mesh,
        compiler_params=pltpu.CompilerParams(
            needs_layout_passes=False, use_tc_tiling_on_sc=False),
        scratch_types=[
            pltpu.VMEM((_C, _D), jnp.float32),
            pltpu.VMEM((_BW, _D), jnp.float32),
            pltpu.VMEM((_BH,), jnp.int32),
            pltpu.VMEM((_BW,), jnp.int32),
            pltpu.VMEM((_CP,), jnp.float32),
            pltpu.VMEM((_NS, _CP), jnp.float32),
            pltpu.VMEM((_CP,), jnp.float32),
            pltpu.VMEM((_CP,), jnp.float32),
            pltpu.VMEM((_L,), jnp.float32),
            pltpu.VMEM((_NS, _L), jnp.float32),
            pltpu.VMEM_SHARED((_NS, _CP), jnp.float32),
            pltpu.VMEM_SHARED((_NS, _L), jnp.float32),
            pltpu.SemaphoreType.DMA,
            pltpu.SemaphoreType.DMA,
        ],
    )(features, labels, centers)
    return jnp.sum(out)


def kernel(features, labels, centers):
    labels = labels.reshape(-1).astype(jnp.int32)
    return _center_loss_sc(features, labels, centers)


# speedup vs baseline: 1.9056x; 1.0304x over previous
import functools

import jax
import jax.numpy as jnp
from jax import lax
from jax.experimental import pallas as pl
from jax.experimental.pallas import tpu as pltpu
from jax.experimental.pallas import tpu_sc as plsc

_B = 16384
_D = 64
_C = 1000
_CP = 1008
_L = 16
_NC = 2
_NS = 16
_NW = _NC * _NS
_BW = _B // _NW
_BH = _B // _NS


def _body(features_hbm, labels_hbm, centers_hbm, out_hbm,
          cent_v, feat_v, lab_hist_v, lab_my_v, hist_v, hist_all_v,
          tot_v, inv_v, out_v, part_v,
          hist_stage_s, part_stage_s, sem_c, sem_f):
    cid = lax.axis_index("c")
    sid = lax.axis_index("s")
    wid = cid * _NS + sid

    cp_c = pltpu.async_copy(centers_hbm, cent_v, sem_c)
    cp_f = pltpu.async_copy(
        features_hbm.at[pl.ds(wid * _BW, _BW)], feat_v, sem_f)

    with jax.named_scope("ph1_labels_dma"):
        pltpu.sync_copy(labels_hbm.at[pl.ds(sid * _BH, _BH)], lab_hist_v)

    with jax.named_scope("ph1_hist"):
        def zero_hist(k, _):
            hist_v[pl.ds(k * _L, _L)] = jnp.zeros((_L,), jnp.float32)
            return 0
        lax.fori_loop(0, _CP // _L, zero_hist, 0)

        ones = jnp.ones((_L,), jnp.float32)

        def hist_step(i, _):
            idx = lab_hist_v[pl.ds(i * _L, _L)]
            plsc.addupdate_scatter(hist_v, [idx], ones)
            return 0
        lax.fori_loop(0, _BH // _L, hist_step, 0)

    with jax.named_scope("ph1_allreduce"):
        pltpu.sync_copy(hist_v, hist_stage_s.at[sid])
        plsc.subcore_barrier()
        pltpu.sync_copy(hist_stage_s, hist_all_v)

        def sum_hist(k, _):
            sl = pl.ds(k * _L, _L)
            acc = hist_all_v[0, sl]
            def add_row(r, a):
                return a + hist_all_v[r, sl]
            tot_v[sl] = lax.fori_loop(1, _NS, add_row, acc)
            return 0
        lax.fori_loop(0, _CP // _L, sum_hist, 0)

        def inv_step(k, _):
            sl = pl.ds(k * _L, _L)
            n = tot_v[sl]
            inv_v[sl] = jnp.where(n > 0.0, 1.0 / n, 0.0)
            return 0
        lax.fori_loop(0, _CP // _L, inv_step, 0)

    with jax.named_scope("ph2_dma_wait"):
        cp_c.wait()
        cp_f.wait()
        pltpu.sync_copy(labels_hbm.at[pl.ds(wid * _BW, _BW)], lab_my_v)

    with jax.named_scope("ph2_main"):
        zero = jnp.zeros((_L,), jnp.float32)

        @plsc.parallel_loop(0, _BW // _L, carry=(zero, zero, zero, zero))
        def acc_loop(i, carry):
            a0, a1, a2, a3 = carry
            rows = i * _L + lax.iota(jnp.int32, _L)
            idx = lab_my_v[pl.ds(i * _L, _L)]
            inv16 = plsc.load_gather(inv_v, [idx])
            ps = [zero, zero, zero, zero]
            for k in range(_D):
                kcol = jnp.full((_L,), k, jnp.int32)
                f = plsc.load_gather(feat_v, [rows, kcol])
                c = plsc.load_gather(cent_v, [idx, kcol])
                dlt = f - c
                ps[k % 4] = ps[k % 4] + dlt * dlt
            return (a0 + ps[0] * inv16, a1 + ps[1] * inv16,
                    a2 + ps[2] * inv16, a3 + ps[3] * inv16)
        a0, a1, a2, a3 = acc_loop
        acc = (a0 + a1) + (a2 + a3)

    out_v[...] = acc
    pltpu.sync_copy(out_v, part_stage_s.at[sid])
    plsc.subcore_barrier()

    @pl.when(sid == 0)
    def _():
        pltpu.sync_copy(part_stage_s, part_v)
        def add_part(r, a):
            return a + part_v[r, :]
        tot = lax.fori_loop(1, _NS, add_part, part_v[0, :])
        out_v[...] = tot * (1.0 / (_D * _B))
        pltpu.sync_copy(out_v, out_hbm.at[cid])


@jax.jit
def _center_loss_sc(features, labels, centers):
    mesh = plsc.VectorSubcoreMesh(core_axis_name="c", subcore_axis_name="s")
    out = pl.kernel(
        _body,
        out_type=jax.ShapeDtypeStruct((_NC, _L), jnp.float32),
        mesh=mesh,
        compiler_params=pltpu.CompilerParams(
            needs_layout_passes=False, use_tc_tiling_on_sc=False),
        scratch_types=[
            pltpu.VMEM((_C, _D), jnp.float32),
            pltpu.VMEM((_BW, _D), jnp.float32),
            pltpu.VMEM((_BH,), jnp.int32),
            pltpu.VMEM((_BW,), jnp.int32),
            pltpu.VMEM((_CP,), jnp.float32),
            pltpu.VMEM((_NS, _CP), jnp.float32),
            pltpu.VMEM((_CP,), jnp.float32),
            pltpu.VMEM((_CP,), jnp.float32),
            pltpu.VMEM((_L,), jnp.float32),
            pltpu.VMEM((_NS, _L), jnp.float32),
            pltpu.VMEM_SHARED((_NS, _CP), jnp.float32),
            pltpu.VMEM_SHARED((_NS, _L), jnp.float32),
            pltpu.SemaphoreType.DMA,
            pltpu.SemaphoreType.DMA,
        ],
    )(features, labels, centers)
    return jnp.sum(out)


def kernel(features, labels, centers):
    labels = labels.reshape(-1).astype(jnp.int32)
    return _center_loss_sc(features, labels, centers)


# speedup vs baseline: 2.7936x; 1.4660x over previous
import functools

import jax
import jax.numpy as jnp
from jax import lax
from jax.experimental import pallas as pl
from jax.experimental.pallas import tpu as pltpu
from jax.experimental.pallas import tpu_sc as plsc

_B = 16384
_D = 64
_C = 1000
_CP = 1008
_L = 16
_NC = 2
_NS = 16
_NW = _NC * _NS
_BW = _B // _NW
_BH = _B // _NS


def _body(features_hbm, labels_hbm, centers_hbm, out_hbm,
          cent_v, feat_v, lab_hist_v, lab_my_v, hist_v, hist_all_v,
          tot_v, inv_v, out_v, part_v,
          hist_stage_s, part_stage_s, sem_c, sem_f):
    cid = lax.axis_index("c")
    sid = lax.axis_index("s")
    wid = cid * _NS + sid

    cp_c = pltpu.async_copy(centers_hbm, cent_v, sem_c)
    cp_f = pltpu.async_copy(
        features_hbm.at[pl.ds(wid * _BW, _BW)], feat_v, sem_f)

    with jax.named_scope("ph1_labels_dma"):
        pltpu.sync_copy(labels_hbm.at[pl.ds(sid * _BH, _BH)], lab_hist_v)

    with jax.named_scope("ph1_hist"):
        def zero_hist(k, _):
            hist_v[pl.ds(k * _L, _L)] = jnp.zeros((_L,), jnp.float32)
            return 0
        lax.fori_loop(0, _CP // _L, zero_hist, 0)

        ones = jnp.ones((_L,), jnp.float32)

        def hist_step(i, _):
            idx = lab_hist_v[pl.ds(i * _L, _L)]
            plsc.addupdate_scatter(hist_v, [idx], ones)
            return 0
        lax.fori_loop(0, _BH // _L, hist_step, 0)

    with jax.named_scope("ph1_allreduce"):
        pltpu.sync_copy(hist_v, hist_stage_s.at[sid])
        plsc.subcore_barrier()
        pltpu.sync_copy(hist_stage_s, hist_all_v)

        def sum_hist(k, _):
            sl = pl.ds(k * _L, _L)
            acc = hist_all_v[0, sl]
            def add_row(r, a):
                return a + hist_all_v[r, sl]
            tot_v[sl] = lax.fori_loop(1, _NS, add_row, acc)
            return 0
        lax.fori_loop(0, _CP // _L, sum_hist, 0)

        def inv_step(k, _):
            sl = pl.ds(k * _L, _L)
            n = tot_v[sl]
            inv_v[sl] = jnp.where(n > 0.0, 1.0 / n, 0.0)
            return 0
        lax.fori_loop(0, _CP // _L, inv_step, 0)

    with jax.named_scope("ph2_dma_wait"):
        cp_c.wait()
        cp_f.wait()
        pltpu.sync_copy(labels_hbm.at[pl.ds(wid * _BW, _BW)], lab_my_v)

    with jax.named_scope("ph2_main"):
        zero = jnp.zeros((_L,), jnp.float32)

        @plsc.parallel_loop(0, _BW // _L, carry=(zero, zero, zero, zero))
        def acc_loop(i, carry):
            accs = list(carry)
            idx = lab_my_v[pl.ds(i * _L, _L)]
            inv16 = plsc.load_gather(inv_v, [idx])
            for j in range(_L):
                l = idx[j]
                inv_j = inv16[j]
                sq = []
                for k in range(_D // _L):
                    sl = pl.ds(k * _L, _L)
                    dlt = feat_v[i * _L + j, sl] - cent_v[l, sl]
                    sq.append(dlt * dlt)
                s = (sq[0] + sq[1]) + (sq[2] + sq[3])
                accs[j % 4] = accs[j % 4] + s * inv_j
            return tuple(accs)
        a0, a1, a2, a3 = acc_loop
        acc = (a0 + a1) + (a2 + a3)

    out_v[...] = acc
    pltpu.sync_copy(out_v, part_stage_s.at[sid])
    plsc.subcore_barrier()

    @pl.when(sid == 0)
    def _():
        pltpu.sync_copy(part_stage_s, part_v)
        def add_part(r, a):
            return a + part_v[r, :]
        tot = lax.fori_loop(1, _NS, add_part, part_v[0, :])
        out_v[...] = tot * (1.0 / (_D * _B))
        pltpu.sync_copy(out_v, out_hbm.at[cid])


@jax.jit
def _center_loss_sc(features, labels, centers):
    mesh = plsc.VectorSubcoreMesh(core_axis_name="c", subcore_axis_name="s")
    out = pl.kernel(
        _body,
        out_type=jax.ShapeDtypeStruct((_NC, _L), jnp.float32),
        mesh=mesh,
        compiler_params=pltpu.CompilerParams(
            needs_layout_passes=False, use_tc_tiling_on_sc=False),
        scratch_types=[
            pltpu.VMEM((_C, _D), jnp.float32),
            pltpu.VMEM((_BW, _D), jnp.float32),
            pltpu.VMEM((_BH,), jnp.int32),
            pltpu.VMEM((_BW,), jnp.int32),
            pltpu.VMEM((_CP,), jnp.float32),
            pltpu.VMEM((_NS, _CP), jnp.float32),
            pltpu.VMEM((_CP,), jnp.float32),
            pltpu.VMEM((_CP,), jnp.float32),
            pltpu.VMEM((_L,), jnp.float32),
            pltpu.VMEM((_NS, _L), jnp.float32),
            pltpu.VMEM_SHARED((_NS, _CP), jnp.float32),
            pltpu.VMEM_SHARED((_NS, _L), jnp.float32),
            pltpu.SemaphoreType.DMA,
            pltpu.SemaphoreType.DMA,
        ],
    )(features, labels, centers)
    return jnp.sum(out)


def kernel(features, labels, centers):
    labels = labels.reshape(-1).astype(jnp.int32)
    return _center_loss_sc(features, labels, centers)
